# Initial kernel scaffold; baseline (speedup 1.0000x reference)
import jax
import jax.numpy as jnp
from jax.experimental import pallas as pl


def kernel(x, edge_index, W11, b11, W12, b12, W21, b21, W22, b22, Wh, bh, eps1, eps2):
    raise NotImplementedError("write your pallas kernel here")


# speedup vs baseline: 7.1360x; 7.1360x over previous
import functools

import jax
import jax.numpy as jnp
from jax import lax
from jax.experimental import pallas as pl
from jax.experimental.pallas import tpu as pltpu
from jax.experimental.pallas import tpu_sc as plsc

N = 10000
D = 256
E = 160000
LUB_EPS = 0.0001

NC = 2
NS = 16
L = 16

NP = 10496
ROWS_PER_TILE = NP // NS
EPAD = 163840
CHUNK = 128
EPT_A = EPAD // NS
NCH_A = EPT_A // CHUNK
EPT_D = EPAD // (NC * NS)
NCH_D = EPT_D // CHUNK

_mesh = plsc.VectorSubcoreMesh(
    core_axis_name="c", subcore_axis_name="s", num_cores=NC, num_subcores=NS)


@functools.partial(
    pl.kernel,
    out_type=jax.ShapeDtypeStruct((NC, NP, 128), jnp.float32),
    mesh=_mesh,
    scratch_types=[
        pltpu.VMEM_SHARED((NP, 128), jnp.float32),
        pltpu.VMEM((NCH_A, CHUNK), jnp.int32),
        pltpu.VMEM((NCH_A, CHUNK), jnp.int32),
        pltpu.VMEM((CHUNK, 128), jnp.float32),
        pltpu.SemaphoreType.DMA,
    ],
)
def _segsum(x2, src2x, dstA, zrows, agg, acc, src_v, dst_v, rowbuf, sem):
    c = lax.axis_index("c")
    s = lax.axis_index("s")
    r0 = s * ROWS_PER_TILE
    pltpu.sync_copy(zrows, acc.at[pl.ds(r0, ROWS_PER_TILE)])
    pltpu.sync_copy(src2x.at[c, s], src_v)
    pltpu.sync_copy(dstA.at[s], dst_v)
    plsc.subcore_barrier()

    def chunk(j, carry):
        pltpu.async_copy(x2.at[src_v.at[j]], rowbuf, sem).wait()
        pltpu.sync_copy(rowbuf, acc.at[dst_v.at[j]], add=True)
        return carry

    lax.fori_loop(0, NCH_A, chunk, 0)
    plsc.subcore_barrier()
    pltpu.sync_copy(acc.at[pl.ds(r0, ROWS_PER_TILE)],
                    agg.at[c, pl.ds(r0, ROWS_PER_TILE)])


@functools.partial(
    pl.kernel,
    out_type=[jax.ShapeDtypeStruct((NC * NS, NCH_D, CHUNK), jnp.float32),
              jax.ShapeDtypeStruct((NC * NS, NCH_D, CHUNK), jnp.float32)],
    mesh=_mesh,
    scratch_types=[
        pltpu.VMEM((2 * N,), jnp.float32),
        pltpu.VMEM((NCH_D, CHUNK), jnp.int32),
        pltpu.VMEM((NCH_D, CHUNK), jnp.int32),
        pltpu.VMEM((NCH_D, CHUNK), jnp.float32),
        pltpu.VMEM((NCH_D, CHUNK), jnp.float32),
    ],
    compiler_params=pltpu.CompilerParams(needs_layout_passes=False),
)
def _lub(fvf, s2, d2, exo, eyo, fv_v, s_v, d_v, ex_v, ey_v):
    c = lax.axis_index("c")
    s = lax.axis_index("s")
    w = s * NC + c
    pltpu.sync_copy(fvf, fv_v)
    pltpu.sync_copy(s2.at[w], s_v)
    pltpu.sync_copy(d2.at[w], d_v)

    def step(j, carry):
        for k in range(CHUNK // L):
            sl = pl.ds(k * L, L)
            si = s_v[j, sl]
            di = d_v[j, sl]
            ex = jnp.maximum(plsc.load_gather(fv_v, [si]),
                             plsc.load_gather(fv_v, [di])) + LUB_EPS
            ey = jnp.maximum(plsc.load_gather(fv_v, [si + 1]),
                             plsc.load_gather(fv_v, [di + 1])) + LUB_EPS
            ex_v[j, sl] = ex
            ey_v[j, sl] = ey
        return carry

    lax.fori_loop(0, NCH_D, step, 0)
    pltpu.sync_copy(ex_v, exo.at[w])
    pltpu.sync_copy(ey_v, eyo.at[w])


R = 400


def _leaky(h):
    return jnp.where(h >= 0, h, 0.01 * h)


def _mlp_body(scale_ref, x_ref, a_ref, W1_ref, b1_ref, W2_ref, b2_ref, o_ref):
    t = x_ref[...] * scale_ref[0, 0] + jnp.concatenate(
        [a_ref[0], a_ref[1]], axis=-1)
    h = jnp.dot(t, W1_ref[...], preferred_element_type=jnp.float32)
    h = _leaky(h + b1_ref[...])
    h = jnp.dot(h, W2_ref[...], preferred_element_type=jnp.float32)
    o_ref[...] = _leaky(h + b2_ref[...])


_mlp = pl.pallas_call(
    _mlp_body,
    grid=(N // R,),
    in_specs=[
        pl.BlockSpec((1, 1), lambda i: (0, 0)),
        pl.BlockSpec((R, D), lambda i: (i, 0)),
        pl.BlockSpec((2, R, 128), lambda i: (0, i, 0)),
        pl.BlockSpec((D, D), lambda i: (0, 0)),
        pl.BlockSpec((1, D), lambda i: (0, 0)),
        pl.BlockSpec((D, D), lambda i: (0, 0)),
        pl.BlockSpec((1, D), lambda i: (0, 0)),
    ],
    out_specs=pl.BlockSpec((R, D), lambda i: (i, 0)),
    out_shape=jax.ShapeDtypeStruct((N, D), jnp.float32),
)


def _head_body(scale_ref, x_ref, a_ref, W1_ref, b1_ref, W2_ref, b2_ref,
               Wh_ref, bh_ref, o_ref):
    t = x_ref[...] * scale_ref[0, 0] + jnp.concatenate(
        [a_ref[0], a_ref[1]], axis=-1)
    h = jnp.dot(t, W1_ref[...], preferred_element_type=jnp.float32)
    h = _leaky(h + b1_ref[...])
    h = jnp.dot(h, W2_ref[...], preferred_element_type=jnp.float32)
    h = _leaky(h + b2_ref[...])
    f = jnp.dot(h, Wh_ref[...], preferred_element_type=jnp.float32)
    o_ref[...] = jax.nn.sigmoid(f + bh_ref[...])


_head = pl.pallas_call(
    _head_body,
    grid=(N // R,),
    in_specs=[
        pl.BlockSpec((1, 1), lambda i: (0, 0)),
        pl.BlockSpec((R, D), lambda i: (i, 0)),
        pl.BlockSpec((2, R, 128), lambda i: (0, i, 0)),
        pl.BlockSpec((D, D), lambda i: (0, 0)),
        pl.BlockSpec((1, D), lambda i: (0, 0)),
        pl.BlockSpec((D, D), lambda i: (0, 0)),
        pl.BlockSpec((1, D), lambda i: (0, 0)),
        pl.BlockSpec((D, 2), lambda i: (0, 0)),
        pl.BlockSpec((1, 2), lambda i: (0, 0)),
    ],
    out_specs=pl.BlockSpec((R, 2), lambda i: (i, 0)),
    out_shape=jax.ShapeDtypeStruct((N, 2), jnp.float32),
)


def kernel(x, edge_index, W11, b11, W12, b12, W21, b21, W22, b22, Wh, bh,
           eps1, eps2):
    src = edge_index[0]
    dst = edge_index[1]
    pad = EPAD - E
    srcp = jnp.concatenate([src, jnp.zeros((pad,), jnp.int32)])
    dstp_a = jnp.concatenate([dst, jnp.full((pad,), N, jnp.int32)])
    dstp_d = jnp.concatenate([dst, jnp.zeros((pad,), jnp.int32)])

    s2 = 2 * srcp
    src2x = jnp.stack([s2, s2 + 1]).reshape(2, NS, NCH_A, CHUNK)
    dstA = dstp_a.reshape(NS, NCH_A, CHUNK)
    s2D = s2.reshape(NC * NS, NCH_D, CHUNK)
    d2D = (2 * dstp_d).reshape(NC * NS, NCH_D, CHUNK)
    zrows = jnp.zeros((ROWS_PER_TILE, 128), jnp.float32)

    agg1 = _segsum(x.reshape(2 * N, 128), src2x, dstA, zrows)
    h1 = _mlp((1.0 + eps1).reshape(1, 1), x, agg1,
              W11, b11.reshape(1, D), W12, b12.reshape(1, D))
    agg2 = _segsum(h1.reshape(2 * N, 128), src2x, dstA, zrows)
    fv = _head((1.0 + eps2).reshape(1, 1), h1, agg2,
               W21, b21.reshape(1, D), W22, b22.reshape(1, D),
               Wh, bh.reshape(1, 2))
    ex, ey = _lub(fv.reshape(-1), s2D, d2D)
    fe = jnp.stack([ex.reshape(-1)[:E], ey.reshape(-1)[:E]], axis=1)
    return jnp.concatenate([fv, fe], axis=0)
